# Initial kernel scaffold; baseline (speedup 1.0000x reference)
import jax
import jax.numpy as jnp
from jax.experimental import pallas as pl


def kernel(z_src, z_tgt, edge_label_index):
    raise NotImplementedError("write your pallas kernel here")


# speedup vs baseline: 2.0112x; 2.0112x over previous
import functools

import jax
import jax.numpy as jnp
from jax import lax
from jax.experimental import pallas as pl
from jax.experimental.pallas import tpu as pltpu, tpu_sc as plsc

N_NODES = 10000
D = 128
N_EDGES = 320000
NC = 2
NS = 16
NW = NC * NS
EPW = N_EDGES // NW
C = 80
R = EPW // C
NV = D // 16

_GDN = lax.GatherDimensionNumbers(
    offset_dims=(), collapsed_slice_dims=(0,), start_index_map=(0,))


def _rotate(x, idx2d):
    return lax.gather(x, idx2d, _GDN, slice_sizes=(1,),
                      mode=lax.GatherScatterMode.PROMISE_IN_BOUNDS)


def _edge_decoder_body(zsrc_hbm, ztgt_hbm, sidx_hbm, tidx_hbm, out_hbm,
                       sidx_v, tidx_v, srows_v, trows_v, out_v, sem_s, sem_t):
    wid = lax.axis_index("s") * NC + lax.axis_index("c")

    def round_body(r, _):
        base = wid * EPW + r * C
        pltpu.sync_copy(sidx_hbm.at[pl.ds(base, C)], sidx_v)
        pltpu.sync_copy(tidx_hbm.at[pl.ds(base, C)], tidx_v)
        cp_s = pltpu.async_copy(zsrc_hbm.at[sidx_v], srows_v, sem_s)
        cp_t = pltpu.async_copy(ztgt_hbm.at[tidx_v], trows_v, sem_t)
        cp_s.wait()
        cp_t.wait()
        lane = lax.iota(jnp.int32, 16)
        rot = [((lane + sh) % 16).reshape(16, 1) for sh in (8, 4, 2, 1)]
        for g in range(C // 16):
            scores = jnp.zeros((16,), jnp.float32)
            for j in range(16):
                e = 16 * g + j
                acc = srows_v[e, pl.ds(0, 16)] * trows_v[e, pl.ds(0, 16)]
                for k in range(1, NV):
                    acc = acc + srows_v[e, pl.ds(16 * k, 16)] * trows_v[e, pl.ds(16 * k, 16)]
                for p in rot:
                    acc = acc + _rotate(acc, p)
                scores = jnp.where(lane == j, acc, scores)
            out_v[pl.ds(16 * g, 16)] = 1.0 / (1.0 + jnp.exp(-scores))
        pltpu.sync_copy(out_v, out_hbm.at[pl.ds(base, C)])
        return ()

    lax.fori_loop(0, R, round_body, (), unroll=False)


@jax.jit
def _edge_decoder(z_src, z_tgt, src_idx, tgt_idx):
    mesh = plsc.VectorSubcoreMesh(core_axis_name="c", subcore_axis_name="s",
                                  num_cores=NC, num_subcores=NS)
    fn = pl.kernel(
        _edge_decoder_body,
        out_type=jax.ShapeDtypeStruct((N_EDGES,), jnp.float32),
        mesh=mesh,
        scratch_types=[
            pltpu.VMEM((C,), jnp.int32),
            pltpu.VMEM((C,), jnp.int32),
            pltpu.VMEM((C, D), jnp.float32),
            pltpu.VMEM((C, D), jnp.float32),
            pltpu.VMEM((C,), jnp.float32),
            pltpu.SemaphoreType.DMA,
            pltpu.SemaphoreType.DMA,
        ],
    )
    return fn(z_src, z_tgt, src_idx, tgt_idx)


def kernel(z_src, z_tgt, edge_label_index):
    src_idx = edge_label_index[0].astype(jnp.int32)
    tgt_idx = edge_label_index[1].astype(jnp.int32)
    return _edge_decoder(z_src, z_tgt, src_idx, tgt_idx)


# speedup vs baseline: 4.2293x; 2.1029x over previous
import functools

import jax
import jax.numpy as jnp
from jax import lax
from jax.experimental import pallas as pl
from jax.experimental.pallas import tpu as pltpu, tpu_sc as plsc

N_NODES = 10000
D = 128
N_EDGES = 320000
NC = 2
NS = 16
NW = NC * NS
EPW = N_EDGES // NW
C = 80
R = EPW // C
NV = D // 16

_GDN = lax.GatherDimensionNumbers(
    offset_dims=(), collapsed_slice_dims=(0,), start_index_map=(0,))


def _rotate(x, idx2d):
    return lax.gather(x, idx2d, _GDN, slice_sizes=(1,),
                      mode=lax.GatherScatterMode.PROMISE_IN_BOUNDS)


def _edge_decoder_body(zsrc_hbm, ztgt_hbm, sidx_hbm, tidx_hbm, out_hbm,
                       sidx_v, tidx_v, srows, trows, out_v, sems):
    wid = lax.axis_index("s") * NC + lax.axis_index("c")
    wbase = wid * EPW
    pltpu.sync_copy(sidx_hbm.at[pl.ds(wbase, EPW)], sidx_v)
    pltpu.sync_copy(tidx_hbm.at[pl.ds(wbase, EPW)], tidx_v)

    def fire(chunk, buf):
        off = chunk * C
        pltpu.async_copy(zsrc_hbm.at[sidx_v.at[pl.ds(off, C)]],
                         srows.at[buf], sems.at[2 * buf])
        pltpu.async_copy(ztgt_hbm.at[tidx_v.at[pl.ds(off, C)]],
                         trows.at[buf], sems.at[2 * buf + 1])

    def drain(buf):
        pltpu.make_async_copy(zsrc_hbm.at[pl.ds(0, C)], srows.at[buf],
                              sems.at[2 * buf]).wait()
        pltpu.make_async_copy(ztgt_hbm.at[pl.ds(0, C)], trows.at[buf],
                              sems.at[2 * buf + 1]).wait()

    lane = lax.iota(jnp.int32, 16)
    rot = [((lane + sh) % 16).reshape(16, 1) for sh in (8, 4, 2, 1)]

    def compute(chunk, buf):
        sv = srows.at[buf]
        tv = trows.at[buf]
        brev = [int(f"{p:04b}"[::-1], 2) for p in range(16)]

        def group_body(g, _):
            m = []
            for j in brev:
                e = 16 * g + j
                p = [sv[e, pl.ds(16 * k, 16)] * tv[e, pl.ds(16 * k, 16)]
                     for k in range(NV)]
                while len(p) > 1:
                    p = [p[i] + p[i + 1] for i in range(0, len(p) - 1, 2)] \
                        + ([p[-1]] if len(p) % 2 else [])
                m.append(p[0])
            gsz = 16
            while len(m) > 1:
                half = gsz // 2
                mask = (lane % gsz) < half
                nxt = []
                for i in range(0, len(m), 2):
                    a = m[i] + _rotate(m[i], ((lane + half) % 16).reshape(16, 1))
                    b = m[i + 1] + _rotate(m[i + 1], ((lane - half) % 16).reshape(16, 1))
                    nxt.append(jnp.where(mask, a, b))
                m = nxt
                gsz = half
            scores = m[0]
            sl = pl.ds(chunk * C + 16 * g, 16)
            out_v[sl] = 1.0 / (1.0 + jnp.exp(-scores))
            return ()

        lax.fori_loop(0, C // 16, group_body, (), unroll=False)

    fire(0, 0)

    def pair_body(p, _):
        a = 2 * p
        fire(a + 1, 1)
        drain(0)
        compute(a, 0)
        fire(a + 2, 0)
        drain(1)
        compute(a + 1, 1)
        return ()

    lax.fori_loop(0, (R - 1) // 2, pair_body, (), unroll=False)
    drain(0)
    compute(R - 1, 0)
    pltpu.sync_copy(out_v, out_hbm.at[pl.ds(wbase, EPW)])


@jax.jit
def _edge_decoder(z_src, z_tgt, src_idx, tgt_idx):
    mesh = plsc.VectorSubcoreMesh(core_axis_name="c", subcore_axis_name="s",
                                  num_cores=NC, num_subcores=NS)
    fn = pl.kernel(
        _edge_decoder_body,
        out_type=jax.ShapeDtypeStruct((N_EDGES,), jnp.float32),
        mesh=mesh,
        scratch_types=[
            pltpu.VMEM((EPW,), jnp.int32),
            pltpu.VMEM((EPW,), jnp.int32),
            pltpu.VMEM((2, C, D), jnp.float32),
            pltpu.VMEM((2, C, D), jnp.float32),
            pltpu.VMEM((EPW,), jnp.float32),
            pltpu.SemaphoreType.DMA((4,)),
        ],
    )
    return fn(z_src, z_tgt, src_idx, tgt_idx)


def kernel(z_src, z_tgt, edge_label_index):
    src_idx = edge_label_index[0].astype(jnp.int32)
    tgt_idx = edge_label_index[1].astype(jnp.int32)
    return _edge_decoder(z_src, z_tgt, src_idx, tgt_idx)


# speedup vs baseline: 9.1631x; 2.1666x over previous
import functools

import jax
import jax.numpy as jnp
from jax import lax
from jax.experimental import pallas as pl
from jax.experimental.pallas import tpu as pltpu, tpu_sc as plsc

N_NODES = 10000
D = 128
N_EDGES = 320000
NC = 2
NS = 16
NW = NC * NS
EPW = N_EDGES // NW
C = 80
R = EPW // C
NV = D // 16

_GDN = lax.GatherDimensionNumbers(
    offset_dims=(), collapsed_slice_dims=(0,), start_index_map=(0,))


def _rotate(x, idx2d):
    return lax.gather(x, idx2d, _GDN, slice_sizes=(1,),
                      mode=lax.GatherScatterMode.PROMISE_IN_BOUNDS)


def _edge_decoder_body(zsrc_hbm, ztgt_hbm, sidx_hbm, tidx_hbm, out_hbm,
                       sidx_v, tidx_v, srows, trows, out_v, sems):
    wid = lax.axis_index("s") * NC + lax.axis_index("c")
    wbase = wid * EPW
    pltpu.sync_copy(sidx_hbm.at[pl.ds(wbase, EPW)], sidx_v)
    pltpu.sync_copy(tidx_hbm.at[pl.ds(wbase, EPW)], tidx_v)

    def fire(chunk, buf):
        off = chunk * C
        pltpu.async_copy(zsrc_hbm.at[sidx_v.at[pl.ds(off, C)]],
                         srows.at[buf], sems.at[2 * buf])
        pltpu.async_copy(ztgt_hbm.at[tidx_v.at[pl.ds(off, C)]],
                         trows.at[buf], sems.at[2 * buf + 1])

    def drain(buf):
        pltpu.make_async_copy(zsrc_hbm.at[pl.ds(0, C)], srows.at[buf],
                              sems.at[2 * buf]).wait()
        pltpu.make_async_copy(ztgt_hbm.at[pl.ds(0, C)], trows.at[buf],
                              sems.at[2 * buf + 1]).wait()

    lane = lax.iota(jnp.int32, 16)
    rot = [((lane + sh) % 16).reshape(16, 1) for sh in (8, 4, 2, 1)]

    brev = [int(f"{p:04b}"[::-1], 2) for p in range(16)]
    rotp = [((lane + h) % 16).reshape(16, 1) for h in (8, 4, 2, 1)]
    rotm = [((lane - h) % 16).reshape(16, 1) for h in (8, 4, 2, 1)]
    masks = [(lane % (16 >> l)) < (8 >> l) for l in range(4)]

    def _merge(a, b, lvl):
        fa = a + _rotate(a, rotp[lvl])
        fb = b + _rotate(b, rotm[lvl])
        return jnp.where(masks[lvl], fa, fb)

    def compute(chunk, buf):
        sv = srows.at[buf]
        tv = trows.at[buf]

        def group_body(g, _):
            def edge_body(j, scores):
                e = 16 * g + j
                p = [sv[e, pl.ds(16 * k, 16)] * tv[e, pl.ds(16 * k, 16)]
                     for k in range(NV)]
                while len(p) > 1:
                    p = [p[i] + p[i + 1] for i in range(0, len(p), 2)]
                acc = p[0]
                for r_ in rot:
                    acc = acc + _rotate(acc, r_)
                return jnp.where(lane == j, acc, scores)

            scores = lax.fori_loop(0, 16, edge_body,
                                   jnp.zeros((16,), jnp.float32), unroll=False)
            sl = pl.ds(chunk * C + 16 * g, 16)
            out_v[sl] = 1.0 / (1.0 + jnp.exp(-scores))
            return ()

        lax.fori_loop(0, C // 16, group_body, (), unroll=False)

    fire(0, 0)

    def pair_body(p, _):
        a = 2 * p
        fire(a + 1, 1)
        drain(0)
        compute(a, 0)
        fire(a + 2, 0)
        drain(1)
        compute(a + 1, 1)
        return ()

    lax.fori_loop(0, (R - 1) // 2, pair_body, (), unroll=False)
    drain(0)
    compute(R - 1, 0)
    pltpu.sync_copy(out_v, out_hbm.at[pl.ds(wbase, EPW)])


@jax.jit
def _edge_decoder(z_src, z_tgt, src_idx, tgt_idx):
    mesh = plsc.VectorSubcoreMesh(core_axis_name="c", subcore_axis_name="s",
                                  num_cores=NC, num_subcores=NS)
    fn = pl.kernel(
        _edge_decoder_body,
        out_type=jax.ShapeDtypeStruct((N_EDGES,), jnp.float32),
        mesh=mesh,
        scratch_types=[
            pltpu.VMEM((EPW,), jnp.int32),
            pltpu.VMEM((EPW,), jnp.int32),
            pltpu.VMEM((2, C, D), jnp.float32),
            pltpu.VMEM((2, C, D), jnp.float32),
            pltpu.VMEM((EPW,), jnp.float32),
            pltpu.SemaphoreType.DMA((4,)),
        ],
    )
    return fn(z_src, z_tgt, src_idx, tgt_idx)


def kernel(z_src, z_tgt, edge_label_index):
    src_idx = edge_label_index[0].astype(jnp.int32)
    tgt_idx = edge_label_index[1].astype(jnp.int32)
    return _edge_decoder(z_src, z_tgt, src_idx, tgt_idx)


# speedup vs baseline: 10.1851x; 1.1115x over previous
import jax
import jax.numpy as jnp
from jax import lax
from jax.experimental import pallas as pl
from jax.experimental.pallas import tpu as pltpu, tpu_sc as plsc

N_NODES = 10000
D = 128
N_EDGES = 320000
NC = 2
NS = 16
NW = NC * NS
EPW = N_EDGES // NW
C = 80
R = EPW // C
NV = D // 16

_GDN = lax.GatherDimensionNumbers(
    offset_dims=(), collapsed_slice_dims=(0,), start_index_map=(0,))


def _rotate(x, idx2d):
    return lax.gather(x, idx2d, _GDN, slice_sizes=(1,),
                      mode=lax.GatherScatterMode.PROMISE_IN_BOUNDS)


def _edge_decoder_body(zsrc_hbm, ztgt_hbm, sidx_hbm, tidx_hbm, out_hbm,
                       sidx_b, tidx_b, srows, trows, out_b, zsrc_sh,
                       rsems, isems, osems):
    sid = lax.axis_index("s")
    wid = sid * NC + lax.axis_index("c")
    wbase = wid * EPW
    @pl.when(sid == 0)
    def _():
        pltpu.sync_copy(zsrc_hbm, zsrc_sh)
    plsc.subcore_barrier()

    def clamp_off(chunk):
        return jnp.minimum(chunk, R - 1) * C

    def fire_idx(chunk, b):
        off = wbase + clamp_off(chunk)
        pltpu.async_copy(sidx_hbm.at[pl.ds(off, C)], sidx_b.at[b], isems.at[b])
        pltpu.async_copy(tidx_hbm.at[pl.ds(off, C)], tidx_b.at[b], isems.at[b])

    def drain_idx(b):
        pltpu.make_async_copy(sidx_hbm.at[pl.ds(0, C)], sidx_b.at[b],
                              isems.at[b]).wait()
        pltpu.make_async_copy(tidx_hbm.at[pl.ds(0, C)], tidx_b.at[b],
                              isems.at[b]).wait()

    def fire_rows(b):
        pltpu.async_copy(zsrc_sh.at[sidx_b.at[b]], srows.at[b], rsems.at[2 * b])
        pltpu.async_copy(ztgt_hbm.at[tidx_b.at[b]], trows.at[b],
                         rsems.at[2 * b + 1])

    def drain_rows(b):
        pltpu.make_async_copy(zsrc_hbm.at[pl.ds(0, C)], srows.at[b],
                              rsems.at[2 * b]).wait()
        pltpu.make_async_copy(zsrc_hbm.at[pl.ds(0, C)], trows.at[b],
                              rsems.at[2 * b + 1]).wait()

    def fire_out(chunk, b):
        pltpu.async_copy(out_b.at[b], out_hbm.at[pl.ds(wbase + chunk * C, C)],
                         osems.at[b])

    def drain_out(b):
        pltpu.make_async_copy(out_b.at[b], out_hbm.at[pl.ds(0, C)],
                              osems.at[b]).wait()

    lane = lax.iota(jnp.int32, 16)
    rot = [((lane + sh) % 16).reshape(16, 1) for sh in (8, 4, 2, 1)]

    def compute(b):
        sv = srows.at[b]
        tv = trows.at[b]

        def group_body(g, _):
            def edge_body(j, scores):
                e = 16 * g + j
                p = [sv[e, pl.ds(16 * k, 16)] * tv[e, pl.ds(16 * k, 16)]
                     for k in range(NV)]
                while len(p) > 1:
                    p = [p[i] + p[i + 1] for i in range(0, len(p), 2)]
                acc = p[0]
                for r_ in rot:
                    acc = acc + _rotate(acc, r_)
                return jnp.where(lane == j, acc, scores)

            scores = lax.fori_loop(0, 16, edge_body,
                                   jnp.zeros((16,), jnp.float32), unroll=False)
            out_b[b, pl.ds(16 * g, 16)] = 1.0 / (1.0 + jnp.exp(-scores))
            return ()

        lax.fori_loop(0, C // 16, group_body, (), unroll=False)

    fire_idx(0, 0)
    drain_idx(0)
    fire_rows(0)
    fire_idx(1, 1)
    drain_idx(1)

    def pair_body(p, _):
        a = 2 * p
        fire_rows(1)
        drain_rows(0)
        fire_idx(a + 2, 0)

        @pl.when(p > 0)
        def _():
            drain_out(0)
        compute(0)
        fire_out(a, 0)
        drain_idx(0)
        fire_rows(0)
        drain_rows(1)
        fire_idx(a + 3, 1)

        @pl.when(p > 0)
        def _():
            drain_out(1)
        compute(1)
        fire_out(a + 1, 1)
        drain_idx(1)
        return ()

    lax.fori_loop(0, (R - 1) // 2, pair_body, (), unroll=False)
    drain_rows(0)
    drain_out(0)
    compute(0)
    fire_out(R - 1, 0)
    drain_out(0)
    drain_out(1)


@jax.jit
def _edge_decoder(z_src, z_tgt, src_idx, tgt_idx):
    mesh = plsc.VectorSubcoreMesh(core_axis_name="c", subcore_axis_name="s",
                                  num_cores=NC, num_subcores=NS)
    fn = pl.kernel(
        _edge_decoder_body,
        out_type=jax.ShapeDtypeStruct((N_EDGES,), jnp.float32),
        mesh=mesh,
        scratch_types=[
            pltpu.VMEM((2, C), jnp.int32),
            pltpu.VMEM((2, C), jnp.int32),
            pltpu.VMEM((2, C, D), jnp.float32),
            pltpu.VMEM((2, C, D), jnp.float32),
            pltpu.VMEM((2, C), jnp.float32),
            pltpu.VMEM_SHARED((N_NODES, D), jnp.float32),
            pltpu.SemaphoreType.DMA((4,)),
            pltpu.SemaphoreType.DMA((2,)),
            pltpu.SemaphoreType.DMA((2,)),
        ],
    )
    return fn(z_src, z_tgt, src_idx, tgt_idx)


def kernel(z_src, z_tgt, edge_label_index):
    src_idx = edge_label_index[0].astype(jnp.int32)
    tgt_idx = edge_label_index[1].astype(jnp.int32)
    return _edge_decoder(z_src, z_tgt, src_idx, tgt_idx)


# speedup vs baseline: 11.6888x; 1.1476x over previous
import jax
import jax.numpy as jnp
from jax import lax
from jax.experimental import pallas as pl
from jax.experimental.pallas import tpu as pltpu, tpu_sc as plsc

N_NODES = 10000
D = 128
N_EDGES = 320000
NC = 2
NS = 16
NW = NC * NS
EPW = N_EDGES // NW
C = 80
R = EPW // C
NV = D // 16
NSLOT = 4
NBLK = N_NODES // C

_GDN = lax.GatherDimensionNumbers(
    offset_dims=(), collapsed_slice_dims=(0,), start_index_map=(0,))


def _rotate(x, idx2d):
    return lax.gather(x, idx2d, _GDN, slice_sizes=(1,),
                      mode=lax.GatherScatterMode.PROMISE_IN_BOUNDS)


def _edge_decoder_body(zsrc_hbm, ztgt_hbm, sidx_hbm, tidx_hbm, out_hbm,
                       sidx_b, tidx_b, rows, nsb, ntb, out_b,
                       zsrc_sh, ns_sh, nt_sh,
                       ssems, asems, isems, nsems, osems):
    sid = lax.axis_index("s")
    wid = sid * NC + lax.axis_index("c")
    wbase = wid * EPW

    lane = lax.iota(jnp.int32, 16)
    rot = [((lane + sh) % 16).reshape(16, 1) for sh in (8, 4, 2, 1)]

    def rowsq_sums(view, g):
        def edge_body(j, sums):
            e = 16 * g + j
            p = [view[e, pl.ds(16 * k, 16)] for k in range(NV)]
            p = [x * x for x in p]
            while len(p) > 1:
                p = [p[i] + p[i + 1] for i in range(0, len(p), 2)]
            acc = p[0]
            for r_ in rot:
                acc = acc + _rotate(acc, r_)
            return jnp.where(lane == j, acc, sums)

        return lax.fori_loop(0, 16, edge_body,
                             jnp.zeros((16,), jnp.float32), unroll=False)

    @pl.when(sid == 0)
    def _():
        pltpu.sync_copy(zsrc_hbm, zsrc_sh)

    def nrm_item(i):
        blk = jnp.minimum(sid + 16 * (i // 2), NBLK - 1)
        tbl = i % 2
        return blk, tbl

    def nrm_fire_load(i):
        blk, tbl = nrm_item(i)
        sl = i % NSLOT
        src = zsrc_hbm if tbl == 0 else ztgt_hbm
        pltpu.async_copy(src.at[pl.ds(blk * C, C)], rows.at[sl], ssems.at[sl])

    def nrm_drain_load(sl):
        pltpu.make_async_copy(zsrc_hbm.at[pl.ds(0, C)], rows.at[sl],
                              ssems.at[sl]).wait()

    def nrm_fire_store(i):
        blk, tbl = nrm_item(i)
        sl = i % NSLOT
        dst = ns_sh if tbl == 0 else nt_sh
        pltpu.async_copy(out_b.at[sl], dst.at[pl.ds(blk * C, C)],
                         osems.at[sl])

    def nrm_drain_store(sl):
        pltpu.make_async_copy(out_b.at[sl], out_hbm.at[pl.ds(0, C)],
                              osems.at[sl]).wait()

    for i in range(NSLOT):
        nrm_fire_load(i)
    for i in range(16):
        sl = i % NSLOT
        nrm_drain_load(sl)
        if i >= NSLOT:
            nrm_drain_store(sl)

        def nrm_group(g, _, sl=sl):
            out_b[sl, pl.ds(16 * g, 16)] = rowsq_sums(rows.at[sl], g)
            return ()

        lax.fori_loop(0, C // 16, nrm_group, (), unroll=False)
        if i + NSLOT < 16:
            nrm_fire_load(i + NSLOT)
        nrm_fire_store(i)
    for i in range(12, 16):
        nrm_drain_store(i % NSLOT)

    plsc.subcore_barrier()

    def fire_I(chunk, sl):
        off = wbase + chunk * C
        pltpu.async_copy(sidx_hbm.at[pl.ds(off, C)], sidx_b.at[sl],
                         isems.at[sl])
        pltpu.async_copy(tidx_hbm.at[pl.ds(off, C)], tidx_b.at[sl],
                         isems.at[sl])

    def drain_I(sl):
        pltpu.make_async_copy(sidx_hbm.at[pl.ds(0, C)], sidx_b.at[sl],
                              isems.at[sl]).wait()
        pltpu.make_async_copy(tidx_hbm.at[pl.ds(0, C)], tidx_b.at[sl],
                              isems.at[sl]).wait()

    def fire_S(sl):
        pltpu.async_copy(zsrc_sh.at[sidx_b.at[sl]], rows.at[sl], ssems.at[sl])

    def drain_S(sl):
        pltpu.make_async_copy(zsrc_hbm.at[pl.ds(0, C)], rows.at[sl],
                              ssems.at[sl]).wait()

    def fire_A(sl):
        pltpu.async_copy(ztgt_hbm.at[tidx_b.at[sl]], rows.at[sl],
                         asems.at[sl], add=True)

    def drain_A(sl):
        pltpu.make_async_copy(ztgt_hbm.at[pl.ds(0, C)], rows.at[sl],
                              asems.at[sl]).wait()

    def fire_N(sl):
        pltpu.async_copy(ns_sh.at[sidx_b.at[sl]], nsb.at[sl], nsems.at[sl])
        pltpu.async_copy(nt_sh.at[tidx_b.at[sl]], ntb.at[sl], nsems.at[sl])

    def drain_N(sl):
        pltpu.make_async_copy(out_hbm.at[pl.ds(0, C)], nsb.at[sl],
                              nsems.at[sl]).wait()
        pltpu.make_async_copy(out_hbm.at[pl.ds(0, C)], ntb.at[sl],
                              nsems.at[sl]).wait()

    def fire_O(chunk, sl):
        pltpu.async_copy(out_b.at[sl],
                         out_hbm.at[pl.ds(wbase + chunk * C, C)],
                         osems.at[sl])

    def drain_O(sl):
        pltpu.make_async_copy(out_b.at[sl], out_hbm.at[pl.ds(0, C)],
                              osems.at[sl]).wait()

    def compute(sl):
        def group_body(g, _):
            q = rowsq_sums(rows.at[sl], g)
            gs = pl.ds(16 * g, 16)
            scores = 0.5 * (q - nsb[sl, gs] - ntb[sl, gs])
            out_b[sl, gs] = 1.0 / (1.0 + jnp.exp(-scores))
            return ()

        lax.fori_loop(0, C // 16, group_body, (), unroll=False)

    for x in range(NSLOT):
        fire_I(x, x)
        drain_I(x)
        fire_S(x)
        fire_N(x)
    for x in range(2):
        drain_S(x)
        fire_A(x)

    def body(quad, _):
        n0 = 4 * quad
        for j in range(NSLOT):
            n = n0 + j
            sl = j

            @pl.when(n + 2 < R)
            def _(sl=sl):
                drain_S((sl + 2) % NSLOT)
                fire_A((sl + 2) % NSLOT)

            drain_A(sl)
            drain_N(sl)

            @pl.when(n >= NSLOT)
            def _(sl=sl):
                drain_O(sl)

            @pl.when(n + NSLOT < R)
            def _(n=n, sl=sl):
                fire_I(n + NSLOT, sl)

            compute(sl)
            fire_O(n, sl)

            @pl.when(n + NSLOT < R)
            def _(sl=sl):
                drain_I(sl)
                fire_S(sl)
                fire_N(sl)
        return ()

    lax.fori_loop(0, (R - 1) // NSLOT, body, (), unroll=False)

    drain_A(0)
    drain_N(0)
    drain_O(0)
    compute(0)
    fire_O(R - 1, 0)
    for sl in (1, 2, 3, 0):
        drain_O(sl)


@jax.jit
def _edge_decoder(z_src, z_tgt, src_idx, tgt_idx):
    mesh = plsc.VectorSubcoreMesh(core_axis_name="c", subcore_axis_name="s",
                                  num_cores=NC, num_subcores=NS)
    fn = pl.kernel(
        _edge_decoder_body,
        out_type=jax.ShapeDtypeStruct((N_EDGES,), jnp.float32),
        mesh=mesh,
        scratch_types=[
            pltpu.VMEM((NSLOT, C), jnp.int32),
            pltpu.VMEM((NSLOT, C), jnp.int32),
            pltpu.VMEM((NSLOT, C, D), jnp.float32),
            pltpu.VMEM((NSLOT, C), jnp.float32),
            pltpu.VMEM((NSLOT, C), jnp.float32),
            pltpu.VMEM((NSLOT, C), jnp.float32),
            pltpu.VMEM_SHARED((N_NODES, D), jnp.float32),
            pltpu.VMEM_SHARED((N_NODES,), jnp.float32),
            pltpu.VMEM_SHARED((N_NODES,), jnp.float32),
            pltpu.SemaphoreType.DMA((NSLOT,)),
            pltpu.SemaphoreType.DMA((NSLOT,)),
            pltpu.SemaphoreType.DMA((NSLOT,)),
            pltpu.SemaphoreType.DMA((NSLOT,)),
            pltpu.SemaphoreType.DMA((NSLOT,)),
        ],
    )
    return fn(z_src, z_tgt, src_idx, tgt_idx)


def kernel(z_src, z_tgt, edge_label_index):
    src_idx = edge_label_index[0].astype(jnp.int32)
    tgt_idx = edge_label_index[1].astype(jnp.int32)
    return _edge_decoder(z_src, z_tgt, src_idx, tgt_idx)


# speedup vs baseline: 12.0325x; 1.0294x over previous
import jax
import jax.numpy as jnp
from jax import lax
from jax.experimental import pallas as pl
from jax.experimental.pallas import tpu as pltpu, tpu_sc as plsc

N_NODES = 10000
D = 128
N_EDGES = 320000
NC = 2
NS = 16
NW = NC * NS
EPW = N_EDGES // NW
C = 80
R = EPW // C
NV = D // 16
NSLOT = 4
NBLK = N_NODES // C

_GDN = lax.GatherDimensionNumbers(
    offset_dims=(), collapsed_slice_dims=(0,), start_index_map=(0,))


def _rotate(x, idx2d):
    return lax.gather(x, idx2d, _GDN, slice_sizes=(1,),
                      mode=lax.GatherScatterMode.PROMISE_IN_BOUNDS)


def _edge_decoder_body(zsrc_hbm, ztgt_hbm, sidx_hbm, tidx_hbm, out_hbm,
                       sidx_b, tidx_b, rows, nsb, ntb, out_b,
                       zsrc_sh, ns_sh, nt_sh,
                       ssems, asems, isems, nsems, osems):
    sid = lax.axis_index("s")
    wid = sid * NC + lax.axis_index("c")
    wbase = wid * EPW

    lane = lax.iota(jnp.int32, 16)
    rot = [((lane + sh) % 16).reshape(16, 1) for sh in (8, 4, 2, 1)]

    def rowsq_sums(view, g):
        def edge_body(j, sums):
            e = 16 * g + j
            p = [view[e, pl.ds(16 * k, 16)] for k in range(NV)]
            p = [x * x for x in p]
            while len(p) > 1:
                p = [p[i] + p[i + 1] for i in range(0, len(p), 2)]
            acc = p[0]
            for r_ in rot:
                acc = acc + _rotate(acc, r_)
            return jnp.where(lane == j, acc, sums)

        return lax.fori_loop(0, 16, edge_body,
                             jnp.zeros((16,), jnp.float32), unroll=False)

    def nrm_item(i):
        blk = jnp.minimum(sid + 16 * (i // 2), NBLK - 1)
        tbl = i % 2
        return blk, tbl

    def nrm_fire_load(i):
        blk, tbl = nrm_item(i)
        sl = i % NSLOT
        src = zsrc_hbm if tbl == 0 else ztgt_hbm
        pltpu.async_copy(src.at[pl.ds(blk * C, C)], rows.at[sl], ssems.at[sl])

    def nrm_drain_load(sl):
        pltpu.make_async_copy(zsrc_hbm.at[pl.ds(0, C)], rows.at[sl],
                              ssems.at[sl]).wait()

    def nrm_fire_store(i):
        blk, tbl = nrm_item(i)
        sl = i % NSLOT
        dst = ns_sh if tbl == 0 else nt_sh
        pltpu.async_copy(out_b.at[sl], dst.at[pl.ds(blk * C, C)],
                         osems.at[sl])

    def nrm_drain_store(sl):
        pltpu.make_async_copy(out_b.at[sl], out_hbm.at[pl.ds(0, C)],
                              osems.at[sl]).wait()

    def nrm_fire_stage(i):
        blk, _ = nrm_item(i)
        sl = i % NSLOT
        pltpu.async_copy(rows.at[sl], zsrc_sh.at[pl.ds(blk * C, C)],
                         asems.at[sl])

    def nrm_drain_stage(sl):
        pltpu.make_async_copy(zsrc_hbm.at[pl.ds(0, C)], rows.at[sl],
                              asems.at[sl]).wait()

    for i in range(NSLOT):
        nrm_fire_load(i)
    for i in range(16):
        sl = i % NSLOT
        nrm_drain_load(sl)
        if i >= NSLOT:
            nrm_drain_store(sl)

        def nrm_group(g, _, sl=sl):
            out_b[sl, pl.ds(16 * g, 16)] = rowsq_sums(rows.at[sl], g)
            return ()

        lax.fori_loop(0, C // 16, nrm_group, (), unroll=False)
        if i % 2 == 0:
            nrm_fire_stage(i)
        if i + NSLOT < 16:
            if (i + NSLOT) % 2 == 0:
                nrm_drain_stage(sl)
            nrm_fire_load(i + NSLOT)
        nrm_fire_store(i)
    for i in range(12, 16):
        if i % 2 == 0:
            nrm_drain_stage(i % NSLOT)
        nrm_drain_store(i % NSLOT)

    plsc.subcore_barrier()

    def fire_I(chunk, sl):
        off = wbase + chunk * C
        pltpu.async_copy(sidx_hbm.at[pl.ds(off, C)], sidx_b.at[sl],
                         isems.at[sl])
        pltpu.async_copy(tidx_hbm.at[pl.ds(off, C)], tidx_b.at[sl],
                         isems.at[sl])

    def drain_I(sl):
        pltpu.make_async_copy(sidx_hbm.at[pl.ds(0, C)], sidx_b.at[sl],
                              isems.at[sl]).wait()
        pltpu.make_async_copy(tidx_hbm.at[pl.ds(0, C)], tidx_b.at[sl],
                              isems.at[sl]).wait()

    def fire_S(sl):
        pltpu.async_copy(zsrc_sh.at[sidx_b.at[sl]], rows.at[sl], ssems.at[sl])

    def drain_S(sl):
        pltpu.make_async_copy(zsrc_hbm.at[pl.ds(0, C)], rows.at[sl],
                              ssems.at[sl]).wait()

    def fire_A(sl):
        pltpu.async_copy(ztgt_hbm.at[tidx_b.at[sl]], rows.at[sl],
                         asems.at[sl], add=True)

    def drain_A(sl):
        pltpu.make_async_copy(ztgt_hbm.at[pl.ds(0, C)], rows.at[sl],
                              asems.at[sl]).wait()

    def fire_N(sl):
        pltpu.async_copy(ns_sh.at[sidx_b.at[sl]], nsb.at[sl], nsems.at[sl])
        pltpu.async_copy(nt_sh.at[tidx_b.at[sl]], ntb.at[sl], nsems.at[sl])

    def drain_N(sl):
        pltpu.make_async_copy(out_hbm.at[pl.ds(0, C)], nsb.at[sl],
                              nsems.at[sl]).wait()
        pltpu.make_async_copy(out_hbm.at[pl.ds(0, C)], ntb.at[sl],
                              nsems.at[sl]).wait()

    def fire_O(chunk, sl):
        pltpu.async_copy(out_b.at[sl],
                         out_hbm.at[pl.ds(wbase + chunk * C, C)],
                         osems.at[sl])

    def drain_O(sl):
        pltpu.make_async_copy(out_b.at[sl], out_hbm.at[pl.ds(0, C)],
                              osems.at[sl]).wait()

    def compute(sl):
        def group_body(g, _):
            q = rowsq_sums(rows.at[sl], g)
            gs = pl.ds(16 * g, 16)
            scores = 0.5 * (q - nsb[sl, gs] - ntb[sl, gs])
            out_b[sl, gs] = 1.0 / (1.0 + jnp.exp(-scores))
            return ()

        lax.fori_loop(0, C // 16, group_body, (), unroll=False)

    for x in range(NSLOT):
        fire_I(x, x)
        drain_I(x)
        fire_S(x)
        fire_N(x)
    for x in range(2):
        drain_S(x)
        fire_A(x)

    def body(quad, _):
        n0 = 4 * quad
        for j in range(NSLOT):
            n = n0 + j
            sl = j

            @pl.when(n + 2 < R)
            def _(sl=sl):
                drain_S((sl + 2) % NSLOT)
                fire_A((sl + 2) % NSLOT)

            drain_A(sl)
            drain_N(sl)

            @pl.when(n >= NSLOT)
            def _(sl=sl):
                drain_O(sl)

            @pl.when(n + NSLOT < R)
            def _(n=n, sl=sl):
                fire_I(n + NSLOT, sl)

            compute(sl)
            fire_O(n, sl)

            @pl.when(n + NSLOT < R)
            def _(sl=sl):
                drain_I(sl)
                fire_S(sl)
                fire_N(sl)
        return ()

    lax.fori_loop(0, (R - 1) // NSLOT, body, (), unroll=False)

    drain_A(0)
    drain_N(0)
    drain_O(0)
    compute(0)
    fire_O(R - 1, 0)
    for sl in (1, 2, 3, 0):
        drain_O(sl)


@jax.jit
def _edge_decoder(z_src, z_tgt, src_idx, tgt_idx):
    mesh = plsc.VectorSubcoreMesh(core_axis_name="c", subcore_axis_name="s",
                                  num_cores=NC, num_subcores=NS)
    fn = pl.kernel(
        _edge_decoder_body,
        out_type=jax.ShapeDtypeStruct((N_EDGES,), jnp.float32),
        mesh=mesh,
        scratch_types=[
            pltpu.VMEM((NSLOT, C), jnp.int32),
            pltpu.VMEM((NSLOT, C), jnp.int32),
            pltpu.VMEM((NSLOT, C, D), jnp.float32),
            pltpu.VMEM((NSLOT, C), jnp.float32),
            pltpu.VMEM((NSLOT, C), jnp.float32),
            pltpu.VMEM((NSLOT, C), jnp.float32),
            pltpu.VMEM_SHARED((N_NODES, D), jnp.float32),
            pltpu.VMEM_SHARED((N_NODES,), jnp.float32),
            pltpu.VMEM_SHARED((N_NODES,), jnp.float32),
            pltpu.SemaphoreType.DMA((NSLOT,)),
            pltpu.SemaphoreType.DMA((NSLOT,)),
            pltpu.SemaphoreType.DMA((NSLOT,)),
            pltpu.SemaphoreType.DMA((NSLOT,)),
            pltpu.SemaphoreType.DMA((NSLOT,)),
        ],
    )
    return fn(z_src, z_tgt, src_idx, tgt_idx)


def kernel(z_src, z_tgt, edge_label_index):
    src_idx = edge_label_index[0].astype(jnp.int32)
    tgt_idx = edge_label_index[1].astype(jnp.int32)
    return _edge_decoder(z_src, z_tgt, src_idx, tgt_idx)


# speedup vs baseline: 12.1110x; 1.0065x over previous
import jax
import jax.numpy as jnp
from jax import lax
from jax.experimental import pallas as pl
from jax.experimental.pallas import tpu as pltpu, tpu_sc as plsc

N_NODES = 10000
D = 128
N_EDGES = 320000
NC = 2
NS = 16
NW = NC * NS
EPW = N_EDGES // NW
C = 80
R = EPW // C
NV = D // 16
NSLOT = 4
NBLK = N_NODES // C

_GDN = lax.GatherDimensionNumbers(
    offset_dims=(), collapsed_slice_dims=(0,), start_index_map=(0,))


def _rotate(x, idx2d):
    return lax.gather(x, idx2d, _GDN, slice_sizes=(1,),
                      mode=lax.GatherScatterMode.PROMISE_IN_BOUNDS)


def _edge_decoder_body(zsrc_hbm, ztgt_hbm, sidx_hbm, tidx_hbm, out_hbm,
                       sidx_b, tidx_b, rows, nsb, ntb, out_b,
                       zsrc_sh, ns_sh, nt_sh,
                       ssems, asems, isems, nsems, osems):
    sid = lax.axis_index("s")
    wid = sid * NC + lax.axis_index("c")
    wbase = wid * EPW

    lane = lax.iota(jnp.int32, 16)
    rot = [((lane + sh) % 16).reshape(16, 1) for sh in (8, 4, 2, 1)]

    def rowsq_sums(view, g):
        def edge_body(j, sums):
            e = 16 * g + j
            p = [view[e, pl.ds(16 * k, 16)] for k in range(NV)]
            p = [x * x for x in p]
            while len(p) > 1:
                p = [p[i] + p[i + 1] for i in range(0, len(p), 2)]
            acc = p[0]
            for r_ in rot:
                acc = acc + _rotate(acc, r_)
            return jnp.where(lane == j, acc, sums)

        return lax.fori_loop(0, 16, edge_body,
                             jnp.zeros((16,), jnp.float32), unroll=False)

    def nrm_item(i):
        blk = jnp.minimum(sid + 16 * (i // 2), NBLK - 1)
        tbl = i % 2
        return blk, tbl

    def nrm_fire_load(i):
        blk, tbl = nrm_item(i)
        sl = i % NSLOT
        src = zsrc_hbm if tbl == 0 else ztgt_hbm
        pltpu.async_copy(src.at[pl.ds(blk * C, C)], rows.at[sl], ssems.at[sl])

    def nrm_drain_load(sl):
        pltpu.make_async_copy(zsrc_hbm.at[pl.ds(0, C)], rows.at[sl],
                              ssems.at[sl]).wait()

    def nrm_fire_store(i):
        blk, tbl = nrm_item(i)
        sl = i % NSLOT
        dst = ns_sh if tbl == 0 else nt_sh
        pltpu.async_copy(out_b.at[sl], dst.at[pl.ds(blk * C, C)],
                         osems.at[sl])

    def nrm_drain_store(sl):
        pltpu.make_async_copy(out_b.at[sl], out_hbm.at[pl.ds(0, C)],
                              osems.at[sl]).wait()

    def nrm_fire_stage(i):
        blk, _ = nrm_item(i)
        sl = i % NSLOT
        pltpu.async_copy(rows.at[sl], zsrc_sh.at[pl.ds(blk * C, C)],
                         asems.at[sl])

    def nrm_drain_stage(sl):
        pltpu.make_async_copy(zsrc_hbm.at[pl.ds(0, C)], rows.at[sl],
                              asems.at[sl]).wait()

    for i in range(NSLOT):
        nrm_fire_load(i)
    for i in range(16):
        sl = i % NSLOT
        nrm_drain_load(sl)
        if i >= NSLOT:
            nrm_drain_store(sl)

        def nrm_group(g, _, sl=sl):
            out_b[sl, pl.ds(16 * g, 16)] = rowsq_sums(rows.at[sl], g)
            return ()

        lax.fori_loop(0, C // 16, nrm_group, (), unroll=False)
        if i % 2 == 0:
            nrm_fire_stage(i)
        if i + NSLOT < 16:
            if (i + NSLOT) % 2 == 0:
                nrm_drain_stage(sl)
            nrm_fire_load(i + NSLOT)
        nrm_fire_store(i)
    for i in range(12, 16):
        if i % 2 == 0:
            nrm_drain_stage(i % NSLOT)
        nrm_drain_store(i % NSLOT)

    plsc.subcore_barrier()

    def fire_I(chunk, sl):
        off = wbase + chunk * C
        pltpu.async_copy(sidx_hbm.at[pl.ds(off, C)], sidx_b.at[sl],
                         isems.at[sl])
        pltpu.async_copy(tidx_hbm.at[pl.ds(off, C)], tidx_b.at[sl],
                         isems.at[sl])

    def drain_I(sl):
        pltpu.make_async_copy(sidx_hbm.at[pl.ds(0, C)], sidx_b.at[sl],
                              isems.at[sl]).wait()
        pltpu.make_async_copy(tidx_hbm.at[pl.ds(0, C)], tidx_b.at[sl],
                              isems.at[sl]).wait()

    def fire_S(sl):
        pltpu.async_copy(zsrc_sh.at[sidx_b.at[sl]], rows.at[sl], ssems.at[sl])

    def drain_S(sl):
        pltpu.make_async_copy(zsrc_hbm.at[pl.ds(0, C)], rows.at[sl],
                              ssems.at[sl]).wait()

    def fire_A(sl):
        pltpu.async_copy(ztgt_hbm.at[tidx_b.at[sl]], rows.at[sl],
                         asems.at[sl], add=True)

    def drain_A(sl):
        pltpu.make_async_copy(ztgt_hbm.at[pl.ds(0, C)], rows.at[sl],
                              asems.at[sl]).wait()

    def fire_N(sl):
        pltpu.async_copy(ns_sh.at[sidx_b.at[sl]], nsb.at[sl], nsems.at[sl])
        pltpu.async_copy(nt_sh.at[tidx_b.at[sl]], ntb.at[sl], nsems.at[sl])

    def drain_N(sl):
        pltpu.make_async_copy(out_hbm.at[pl.ds(0, C)], nsb.at[sl],
                              nsems.at[sl]).wait()
        pltpu.make_async_copy(out_hbm.at[pl.ds(0, C)], ntb.at[sl],
                              nsems.at[sl]).wait()

    def fire_O(chunk, sl):
        pltpu.async_copy(out_b.at[sl],
                         out_hbm.at[pl.ds(wbase + chunk * C, C)],
                         osems.at[sl])

    def drain_O(sl):
        pltpu.make_async_copy(out_b.at[sl], out_hbm.at[pl.ds(0, C)],
                              osems.at[sl]).wait()

    def compute(sl):
        def group_body(g, _):
            q = rowsq_sums(rows.at[sl], g)
            gs = pl.ds(16 * g, 16)
            scores = 0.5 * (q - nsb[sl, gs] - ntb[sl, gs])
            out_b[sl, gs] = 1.0 / (1.0 + jnp.exp(-scores))
            return ()

        lax.fori_loop(0, C // 16, group_body, (), unroll=False)

    for x in range(NSLOT):
        fire_I(x, x)
    for x in range(NSLOT):
        drain_I(x)
        fire_S(x)
        fire_N(x)
    for x in range(2):
        drain_S(x)
        fire_A(x)

    def body(quad, _):
        n0 = 4 * quad
        for j in range(NSLOT):
            n = n0 + j
            sl = j

            @pl.when(n + 2 < R)
            def _(sl=sl):
                drain_S((sl + 2) % NSLOT)
                fire_A((sl + 2) % NSLOT)

            drain_A(sl)
            drain_N(sl)

            @pl.when(n >= NSLOT)
            def _(sl=sl):
                drain_O(sl)

            @pl.when(n + NSLOT < R)
            def _(n=n, sl=sl):
                fire_I(n + NSLOT, sl)

            compute(sl)
            fire_O(n, sl)

            @pl.when(n + NSLOT < R)
            def _(sl=sl):
                drain_I(sl)
                fire_S(sl)
                fire_N(sl)
        return ()

    lax.fori_loop(0, (R - 1) // NSLOT, body, (), unroll=False)

    drain_A(0)
    drain_N(0)
    drain_O(0)
    compute(0)
    fire_O(R - 1, 0)
    for sl in (1, 2, 3, 0):
        drain_O(sl)


@jax.jit
def _edge_decoder(z_src, z_tgt, src_idx, tgt_idx):
    mesh = plsc.VectorSubcoreMesh(core_axis_name="c", subcore_axis_name="s",
                                  num_cores=NC, num_subcores=NS)
    fn = pl.kernel(
        _edge_decoder_body,
        out_type=jax.ShapeDtypeStruct((N_EDGES,), jnp.float32),
        mesh=mesh,
        scratch_types=[
            pltpu.VMEM((NSLOT, C), jnp.int32),
            pltpu.VMEM((NSLOT, C), jnp.int32),
            pltpu.VMEM((NSLOT, C, D), jnp.float32),
            pltpu.VMEM((NSLOT, C), jnp.float32),
            pltpu.VMEM((NSLOT, C), jnp.float32),
            pltpu.VMEM((NSLOT, C), jnp.float32),
            pltpu.VMEM_SHARED((N_NODES, D), jnp.float32),
            pltpu.VMEM_SHARED((N_NODES,), jnp.float32),
            pltpu.VMEM_SHARED((N_NODES,), jnp.float32),
            pltpu.SemaphoreType.DMA((NSLOT,)),
            pltpu.SemaphoreType.DMA((NSLOT,)),
            pltpu.SemaphoreType.DMA((NSLOT,)),
            pltpu.SemaphoreType.DMA((NSLOT,)),
            pltpu.SemaphoreType.DMA((NSLOT,)),
        ],
    )
    return fn(z_src, z_tgt, src_idx, tgt_idx)


def kernel(z_src, z_tgt, edge_label_index):
    src_idx = edge_label_index[0].astype(jnp.int32)
    tgt_idx = edge_label_index[1].astype(jnp.int32)
    return _edge_decoder(z_src, z_tgt, src_idx, tgt_idx)


# speedup vs baseline: 12.3377x; 1.0187x over previous
import jax
import jax.numpy as jnp
from jax import lax
from jax.experimental import pallas as pl
from jax.experimental.pallas import tpu as pltpu, tpu_sc as plsc

N_NODES = 10000
D = 128
N_EDGES = 320000
NC = 2
NS = 16
NW = NC * NS
EPW = N_EDGES // NW
C = 80
R = EPW // C
NV = D // 16
NSLOT = 4
NBLK = N_NODES // C

_GDN = lax.GatherDimensionNumbers(
    offset_dims=(), collapsed_slice_dims=(0,), start_index_map=(0,))


def _rotate(x, idx2d):
    return lax.gather(x, idx2d, _GDN, slice_sizes=(1,),
                      mode=lax.GatherScatterMode.PROMISE_IN_BOUNDS)


def _edge_decoder_body(zsrc_hbm, ztgt_hbm, sidx_hbm, tidx_hbm, out_hbm,
                       sidx_b, tidx_b, rows, nsb, ntb, out_b,
                       zsrc_sh, ns_sh, nt_sh,
                       ssems, asems, isems, nsems, osems):
    sid = lax.axis_index("s")
    wid = sid * NC + lax.axis_index("c")
    wbase = wid * EPW

    lane = lax.iota(jnp.int32, 16)
    rot = [((lane + sh) % 16).reshape(16, 1) for sh in (8, 4, 2, 1)]

    def rowsq_sums(view, g):
        def edge_body(j, sums):
            e = 16 * g + j
            p = [view[e, pl.ds(16 * k, 16)] for k in range(NV)]
            p = [x * x for x in p]
            while len(p) > 1:
                p = [p[i] + p[i + 1] for i in range(0, len(p), 2)]
            acc = p[0]
            for r_ in rot:
                acc = acc + _rotate(acc, r_)
            return jnp.where(lane == j, acc, sums)

        return lax.fori_loop(0, 16, edge_body,
                             jnp.zeros((16,), jnp.float32), unroll=False)

    def nrm_item(i):
        blk = jnp.minimum(sid + 16 * (i // 2), NBLK - 1)
        tbl = i % 2
        return blk, tbl

    def nrm_fire_load(i):
        blk, tbl = nrm_item(i)
        sl = i % NSLOT
        src = zsrc_hbm if tbl == 0 else ztgt_hbm
        pltpu.async_copy(src.at[pl.ds(blk * C, C)], rows.at[sl], ssems.at[sl])

    def nrm_drain_load(sl):
        pltpu.make_async_copy(zsrc_hbm.at[pl.ds(0, C)], rows.at[sl],
                              ssems.at[sl]).wait()

    def nrm_fire_store(i):
        blk, tbl = nrm_item(i)
        sl = i % NSLOT
        dst = ns_sh if tbl == 0 else nt_sh
        pltpu.async_copy(out_b.at[sl], dst.at[pl.ds(blk * C, C)],
                         osems.at[sl])

    def nrm_drain_store(sl):
        pltpu.make_async_copy(out_b.at[sl], out_hbm.at[pl.ds(0, C)],
                              osems.at[sl]).wait()

    def nrm_fire_stage(i):
        blk, _ = nrm_item(i)
        sl = i % NSLOT
        pltpu.async_copy(rows.at[sl], zsrc_sh.at[pl.ds(blk * C, C)],
                         asems.at[sl])

    def nrm_drain_stage(sl):
        pltpu.make_async_copy(zsrc_hbm.at[pl.ds(0, C)], rows.at[sl],
                              asems.at[sl]).wait()

    for i in range(NSLOT):
        nrm_fire_load(i)
    for i in range(16):
        sl = i % NSLOT
        nrm_drain_load(sl)
        if i >= NSLOT:
            nrm_drain_store(sl)

        def nrm_group(g, _, sl=sl):
            out_b[sl, pl.ds(16 * g, 16)] = rowsq_sums(rows.at[sl], g)
            return ()

        if i % 2 == 0:
            nrm_fire_stage(i)
        lax.fori_loop(0, C // 16, nrm_group, (), unroll=False)
        if i + NSLOT < 16:
            if (i + NSLOT) % 2 == 0:
                nrm_drain_stage(sl)
            nrm_fire_load(i + NSLOT)
        nrm_fire_store(i)
    for i in range(12, 16):
        if i % 2 == 0:
            nrm_drain_stage(i % NSLOT)
        nrm_drain_store(i % NSLOT)

    plsc.subcore_barrier()

    def fire_I(chunk, sl):
        off = wbase + chunk * C
        pltpu.async_copy(sidx_hbm.at[pl.ds(off, C)], sidx_b.at[sl],
                         isems.at[sl])
        pltpu.async_copy(tidx_hbm.at[pl.ds(off, C)], tidx_b.at[sl],
                         isems.at[sl])

    def drain_I(sl):
        pltpu.make_async_copy(sidx_hbm.at[pl.ds(0, C)], sidx_b.at[sl],
                              isems.at[sl]).wait()
        pltpu.make_async_copy(tidx_hbm.at[pl.ds(0, C)], tidx_b.at[sl],
                              isems.at[sl]).wait()

    def fire_S(sl):
        pltpu.async_copy(zsrc_sh.at[sidx_b.at[sl]], rows.at[sl], ssems.at[sl])

    def drain_S(sl):
        pltpu.make_async_copy(zsrc_hbm.at[pl.ds(0, C)], rows.at[sl],
                              ssems.at[sl]).wait()

    def fire_A(sl):
        pltpu.async_copy(ztgt_hbm.at[tidx_b.at[sl]], rows.at[sl],
                         asems.at[sl], add=True)

    def drain_A(sl):
        pltpu.make_async_copy(ztgt_hbm.at[pl.ds(0, C)], rows.at[sl],
                              asems.at[sl]).wait()

    def fire_N(sl):
        pltpu.async_copy(ns_sh.at[sidx_b.at[sl]], nsb.at[sl], nsems.at[sl])
        pltpu.async_copy(nt_sh.at[tidx_b.at[sl]], ntb.at[sl], nsems.at[sl])

    def drain_N(sl):
        pltpu.make_async_copy(out_hbm.at[pl.ds(0, C)], nsb.at[sl],
                              nsems.at[sl]).wait()
        pltpu.make_async_copy(out_hbm.at[pl.ds(0, C)], ntb.at[sl],
                              nsems.at[sl]).wait()

    def fire_O(chunk, sl):
        pltpu.async_copy(out_b.at[sl],
                         out_hbm.at[pl.ds(wbase + chunk * C, C)],
                         osems.at[sl])

    def drain_O(sl):
        pltpu.make_async_copy(out_b.at[sl], out_hbm.at[pl.ds(0, C)],
                              osems.at[sl]).wait()

    def compute(sl):
        def group_body(g, _):
            q = rowsq_sums(rows.at[sl], g)
            gs = pl.ds(16 * g, 16)
            scores = 0.5 * (q - nsb[sl, gs] - ntb[sl, gs])
            out_b[sl, gs] = 1.0 / (1.0 + jnp.exp(-scores))
            return ()

        lax.fori_loop(0, C // 16, group_body, (), unroll=False)

    for x in range(NSLOT):
        fire_I(x, x)
    for x in range(NSLOT):
        drain_I(x)
        fire_S(x)
        fire_N(x)
    for x in range(2):
        drain_S(x)
        fire_A(x)

    def body(quad, _):
        n0 = 4 * quad
        for j in range(NSLOT):
            n = n0 + j
            sl = j

            @pl.when(n + 2 < R)
            def _(sl=sl):
                drain_S((sl + 2) % NSLOT)
                fire_A((sl + 2) % NSLOT)

            drain_A(sl)
            drain_N(sl)

            @pl.when(n >= NSLOT)
            def _(sl=sl):
                drain_O(sl)

            @pl.when(n + NSLOT < R)
            def _(n=n, sl=sl):
                fire_I(n + NSLOT, sl)

            compute(sl)
            fire_O(n, sl)

            @pl.when(n + NSLOT < R)
            def _(sl=sl):
                drain_I(sl)
                fire_S(sl)
                fire_N(sl)
        return ()

    lax.fori_loop(0, (R - 1) // NSLOT, body, (), unroll=False)

    drain_A(0)
    drain_N(0)
    drain_O(0)
    compute(0)
    fire_O(R - 1, 0)
    for sl in (1, 2, 3, 0):
        drain_O(sl)


@jax.jit
def _edge_decoder(z_src, z_tgt, src_idx, tgt_idx):
    mesh = plsc.VectorSubcoreMesh(core_axis_name="c", subcore_axis_name="s",
                                  num_cores=NC, num_subcores=NS)
    fn = pl.kernel(
        _edge_decoder_body,
        out_type=jax.ShapeDtypeStruct((N_EDGES,), jnp.float32),
        mesh=mesh,
        scratch_types=[
            pltpu.VMEM((NSLOT, C), jnp.int32),
            pltpu.VMEM((NSLOT, C), jnp.int32),
            pltpu.VMEM((NSLOT, C, D), jnp.float32),
            pltpu.VMEM((NSLOT, C), jnp.float32),
            pltpu.VMEM((NSLOT, C), jnp.float32),
            pltpu.VMEM((NSLOT, C), jnp.float32),
            pltpu.VMEM_SHARED((N_NODES, D), jnp.float32),
            pltpu.VMEM_SHARED((N_NODES,), jnp.float32),
            pltpu.VMEM_SHARED((N_NODES,), jnp.float32),
            pltpu.SemaphoreType.DMA((NSLOT,)),
            pltpu.SemaphoreType.DMA((NSLOT,)),
            pltpu.SemaphoreType.DMA((NSLOT,)),
            pltpu.SemaphoreType.DMA((NSLOT,)),
            pltpu.SemaphoreType.DMA((NSLOT,)),
        ],
    )
    return fn(z_src, z_tgt, src_idx, tgt_idx)


def kernel(z_src, z_tgt, edge_label_index):
    src_idx = edge_label_index[0].astype(jnp.int32)
    tgt_idx = edge_label_index[1].astype(jnp.int32)
    return _edge_decoder(z_src, z_tgt, src_idx, tgt_idx)


# speedup vs baseline: 13.2742x; 1.0759x over previous
import jax
import jax.numpy as jnp
from jax import lax
from jax.experimental import pallas as pl
from jax.experimental.pallas import tpu as pltpu, tpu_sc as plsc

N_NODES = 10000
D = 128
N_EDGES = 320000
NC = 2
NS = 16
NW = NC * NS
EPW = N_EDGES // NW
C = 80
R = EPW // C
NV = D // 16
NSLOT = 4
NBLK = N_NODES // C

_GDN = lax.GatherDimensionNumbers(
    offset_dims=(), collapsed_slice_dims=(0,), start_index_map=(0,))


def _rotate(x, idx2d):
    return lax.gather(x, idx2d, _GDN, slice_sizes=(1,),
                      mode=lax.GatherScatterMode.PROMISE_IN_BOUNDS)


def _edge_decoder_body(zsrc_hbm, ztgt_hbm, eidx_hbm, out_hbm,
                       sidx_b, tidx_b, rows, nsb, ntb, out_b,
                       zsrc_sh, ns_sh, nt_sh,
                       ssems, asems, isems, nsems, osems):
    sid = lax.axis_index("s")
    wid = sid * NC + lax.axis_index("c")
    wbase = wid * EPW

    lane = lax.iota(jnp.int32, 16)
    rot = [((lane + sh) % 16).reshape(16, 1) for sh in (8, 4, 2, 1)]

    def rowsq_sums(view, g):
        def edge_body(j, sums):
            e = 16 * g + j
            p = [view[e, pl.ds(16 * k, 16)] for k in range(NV)]
            p = [x * x for x in p]
            while len(p) > 1:
                p = [p[i] + p[i + 1] for i in range(0, len(p), 2)]
            acc = p[0]
            for r_ in rot:
                acc = acc + _rotate(acc, r_)
            return jnp.where(lane == j, acc, sums)

        return lax.fori_loop(0, 16, edge_body,
                             jnp.zeros((16,), jnp.float32), unroll=False)

    def nrm_item(i):
        blk = jnp.minimum(sid + 16 * (i // 2), NBLK - 1)
        tbl = i % 2
        return blk, tbl

    def nrm_fire_load(i):
        blk, tbl = nrm_item(i)
        sl = i % NSLOT
        src = zsrc_hbm if tbl == 0 else ztgt_hbm
        pltpu.async_copy(src.at[pl.ds(blk * C, C)], rows.at[sl], ssems.at[sl])

    def nrm_drain_load(sl):
        pltpu.make_async_copy(zsrc_hbm.at[pl.ds(0, C)], rows.at[sl],
                              ssems.at[sl]).wait()

    def nrm_fire_store(i):
        blk, tbl = nrm_item(i)
        sl = i % NSLOT
        dst = ns_sh if tbl == 0 else nt_sh
        pltpu.async_copy(out_b.at[sl], dst.at[pl.ds(blk * C, C)],
                         osems.at[sl])

    def nrm_drain_store(sl):
        pltpu.make_async_copy(out_b.at[sl], out_hbm.at[pl.ds(0, C)],
                              osems.at[sl]).wait()

    def nrm_fire_stage(i):
        blk, _ = nrm_item(i)
        sl = i % NSLOT
        pltpu.async_copy(rows.at[sl], zsrc_sh.at[pl.ds(blk * C, C)],
                         asems.at[sl])

    def nrm_drain_stage(sl):
        pltpu.make_async_copy(zsrc_hbm.at[pl.ds(0, C)], rows.at[sl],
                              asems.at[sl]).wait()

    for i in range(NSLOT):
        nrm_fire_load(i)
    for i in range(16):
        sl = i % NSLOT
        nrm_drain_load(sl)
        if i >= NSLOT:
            nrm_drain_store(sl)

        def nrm_group(g, _, sl=sl):
            out_b[sl, pl.ds(16 * g, 16)] = rowsq_sums(rows.at[sl], g)
            return ()

        if i % 2 == 0:
            nrm_fire_stage(i)
        lax.fori_loop(0, C // 16, nrm_group, (), unroll=False)
        if i + NSLOT < 16:
            if (i + NSLOT) % 2 == 0:
                nrm_drain_stage(sl)
            nrm_fire_load(i + NSLOT)
        nrm_fire_store(i)
    for i in range(12, 16):
        if i % 2 == 0:
            nrm_drain_stage(i % NSLOT)
        nrm_drain_store(i % NSLOT)

    plsc.subcore_barrier()

    def fire_I(chunk, sl):
        off = wbase + chunk * C
        pltpu.async_copy(eidx_hbm.at[pl.ds(off, C)], sidx_b.at[sl],
                         isems.at[sl])
        pltpu.async_copy(eidx_hbm.at[pl.ds(N_EDGES + off, C)], tidx_b.at[sl],
                         isems.at[sl])

    def drain_I(sl):
        pltpu.make_async_copy(eidx_hbm.at[pl.ds(0, C)], sidx_b.at[sl],
                              isems.at[sl]).wait()
        pltpu.make_async_copy(eidx_hbm.at[pl.ds(0, C)], tidx_b.at[sl],
                              isems.at[sl]).wait()

    def fire_S(sl):
        pltpu.async_copy(zsrc_sh.at[sidx_b.at[sl]], rows.at[sl], ssems.at[sl])

    def drain_S(sl):
        pltpu.make_async_copy(zsrc_hbm.at[pl.ds(0, C)], rows.at[sl],
                              ssems.at[sl]).wait()

    def fire_A(sl):
        pltpu.async_copy(ztgt_hbm.at[tidx_b.at[sl]], rows.at[sl],
                         asems.at[sl], add=True)

    def drain_A(sl):
        pltpu.make_async_copy(ztgt_hbm.at[pl.ds(0, C)], rows.at[sl],
                              asems.at[sl]).wait()

    def fire_N(sl):
        pltpu.async_copy(ns_sh.at[sidx_b.at[sl]], nsb.at[sl], nsems.at[sl])
        pltpu.async_copy(nt_sh.at[tidx_b.at[sl]], ntb.at[sl], nsems.at[sl])

    def drain_N(sl):
        pltpu.make_async_copy(out_hbm.at[pl.ds(0, C)], nsb.at[sl],
                              nsems.at[sl]).wait()
        pltpu.make_async_copy(out_hbm.at[pl.ds(0, C)], ntb.at[sl],
                              nsems.at[sl]).wait()

    def fire_O(chunk, sl):
        pltpu.async_copy(out_b.at[sl],
                         out_hbm.at[pl.ds(wbase + chunk * C, C)],
                         osems.at[sl])

    def drain_O(sl):
        pltpu.make_async_copy(out_b.at[sl], out_hbm.at[pl.ds(0, C)],
                              osems.at[sl]).wait()

    def compute(sl):
        def group_body(g, _):
            q = rowsq_sums(rows.at[sl], g)
            gs = pl.ds(16 * g, 16)
            scores = 0.5 * (q - nsb[sl, gs] - ntb[sl, gs])
            out_b[sl, gs] = 1.0 / (1.0 + jnp.exp(-scores))
            return ()

        lax.fori_loop(0, C // 16, group_body, (), unroll=False)

    for x in range(NSLOT):
        fire_I(x, x)
    for x in range(NSLOT):
        drain_I(x)
        fire_S(x)
        fire_N(x)
    for x in range(2):
        drain_S(x)
        fire_A(x)

    def body(quad, _):
        n0 = 4 * quad
        for j in range(NSLOT):
            n = n0 + j
            sl = j

            @pl.when(n + 2 < R)
            def _(sl=sl):
                drain_S((sl + 2) % NSLOT)
                fire_A((sl + 2) % NSLOT)

            drain_A(sl)
            drain_N(sl)

            @pl.when(n >= NSLOT)
            def _(sl=sl):
                drain_O(sl)

            @pl.when(n + NSLOT < R)
            def _(n=n, sl=sl):
                fire_I(n + NSLOT, sl)

            compute(sl)
            fire_O(n, sl)

            @pl.when(n + NSLOT < R)
            def _(sl=sl):
                drain_I(sl)
                fire_S(sl)
                fire_N(sl)
        return ()

    lax.fori_loop(0, (R - 1) // NSLOT, body, (), unroll=False)

    drain_A(0)
    drain_N(0)
    drain_O(0)
    compute(0)
    fire_O(R - 1, 0)
    for sl in (1, 2, 3, 0):
        drain_O(sl)


@jax.jit
def _edge_decoder(z_src, z_tgt, eidx_flat):
    mesh = plsc.VectorSubcoreMesh(core_axis_name="c", subcore_axis_name="s",
                                  num_cores=NC, num_subcores=NS)
    fn = pl.kernel(
        _edge_decoder_body,
        out_type=jax.ShapeDtypeStruct((N_EDGES,), jnp.float32),
        mesh=mesh,
        scratch_types=[
            pltpu.VMEM((NSLOT, C), jnp.int32),
            pltpu.VMEM((NSLOT, C), jnp.int32),
            pltpu.VMEM((NSLOT, C, D), jnp.float32),
            pltpu.VMEM((NSLOT, C), jnp.float32),
            pltpu.VMEM((NSLOT, C), jnp.float32),
            pltpu.VMEM((NSLOT, C), jnp.float32),
            pltpu.VMEM_SHARED((N_NODES, D), jnp.float32),
            pltpu.VMEM_SHARED((N_NODES,), jnp.float32),
            pltpu.VMEM_SHARED((N_NODES,), jnp.float32),
            pltpu.SemaphoreType.DMA((NSLOT,)),
            pltpu.SemaphoreType.DMA((NSLOT,)),
            pltpu.SemaphoreType.DMA((NSLOT,)),
            pltpu.SemaphoreType.DMA((NSLOT,)),
            pltpu.SemaphoreType.DMA((NSLOT,)),
        ],
    )
    return fn(z_src, z_tgt, eidx_flat)


def kernel(z_src, z_tgt, edge_label_index):
    eidx_flat = edge_label_index.astype(jnp.int32).reshape(2 * N_EDGES)
    return _edge_decoder(z_src, z_tgt, eidx_flat)
